# baseline (speedup 1.0000x reference)
import functools

import jax
import jax.numpy as jnp
from jax import lax
from jax.experimental import pallas as pl
from jax.experimental.pallas import tpu as pltpu
from jax.experimental.pallas import tpu_sc as plsc

_NC = 2
_NS = 16
_NW = _NC * _NS

_CHUNK = 16
_NBUF = 4


def _gather_kernel(total, d_model, n_chunks):
    mesh = plsc.VectorSubcoreMesh(core_axis_name="c", subcore_axis_name="s")
    n_per_w = n_chunks * _CHUNK
    n_groups = n_chunks // _NBUF

    @functools.partial(
        pl.kernel,
        mesh=mesh,
        out_type=jax.ShapeDtypeStruct((total, d_model), jnp.float32),
        scratch_types=[
            pltpu.VMEM((n_chunks, _CHUNK), jnp.int32),
        ]
        + [pltpu.VMEM((_CHUNK, d_model), jnp.float32)] * _NBUF
        + [pltpu.SemaphoreType.DMA] * (2 * _NBUF),
    )
    def k(pe_hbm, idx_hbm, out_hbm, idx_v, *rest):
        bufs = rest[:_NBUF]
        gsems = rest[_NBUF:2 * _NBUF]
        ssems = rest[2 * _NBUF:]

        wid = lax.axis_index("s") * _NC + lax.axis_index("c")
        base = wid * n_per_w
        pltpu.sync_copy(idx_hbm.at[wid], idx_v)

        def g_start(c, j):
            pltpu.async_copy(pe_hbm.at[idx_v.at[c]], bufs[j], gsems[j])

        def g_wait(c, j):
            pltpu.make_async_copy(pe_hbm.at[idx_v.at[c]], bufs[j], gsems[j]).wait()

        def out_ref(c):
            return out_hbm.at[pl.ds(base + c * _CHUNK, _CHUNK)]

        def s_start(c, j):
            pltpu.async_copy(bufs[j], out_ref(c), ssems[j])

        def s_wait(c, j):
            pltpu.make_async_copy(bufs[j], out_ref(c), ssems[j]).wait()

        g_start(0, 0)
        g_start(1, 1)
        g_wait(0, 0); s_start(0, 0); g_start(2, 2)
        g_wait(1, 1); s_start(1, 1); g_start(3, 3)
        g_wait(2, 2); s_start(2, 2); s_wait(0, 0); g_start(4, 0)
        g_wait(3, 3); s_start(3, 3); s_wait(1, 1); g_start(5, 1)

        def body(g, carry):
            c = g * _NBUF
            for j in range(_NBUF):
                jn = (j + 2) % _NBUF
                g_wait(c + j, j)
                s_start(c + j, j)
                s_wait(c + j - 2, jn)
                g_start(c + j + 2, jn)
            return carry

        lax.fori_loop(1, n_groups - 1, body, 0)

        c = n_chunks - _NBUF
        g_wait(c, 0); s_start(c, 0); s_wait(c - 2, 2); g_start(c + 2, 2)
        g_wait(c + 1, 1); s_start(c + 1, 1); s_wait(c - 1, 3); g_start(c + 3, 3)
        g_wait(c + 2, 2); s_start(c + 2, 2)
        g_wait(c + 3, 3); s_start(c + 3, 3)
        s_wait(c, 0)
        s_wait(c + 1, 1)
        s_wait(c + 2, 2)
        s_wait(c + 3, 3)

    return k


def kernel(x, pe):
    batch, seq_len = x.shape
    max_len, d_model = pe.shape
    total = batch * seq_len
    n_per_w = total // _NW
    n_chunks = n_per_w // _CHUNK
    idx = x.reshape(_NW, n_chunks, _CHUNK)
    out = _gather_kernel(total, d_model, n_chunks)(pe, idx)
    return out.reshape(batch, seq_len, d_model)
